# baseline (speedup 1.0000x reference)
import functools

import jax
import jax.numpy as jnp
from jax import lax
from jax.experimental import pallas as pl
from jax.experimental.pallas import tpu as pltpu
from jax.experimental.pallas import tpu_sc as plsc


_NBUF = 6400
_WINDOW = 16


def kernel(edge_index, W):
    E = edge_index.shape[1]
    D = W.shape[1]
    info = plsc.get_sparse_core_info()
    nw = info.num_cores * info.num_subcores
    rows_per_w = D // nw
    n_dma = E // _NBUF
    mesh = plsc.VectorSubcoreMesh(core_axis_name="c", subcore_axis_name="s")

    @functools.partial(
        pl.kernel,
        mesh=mesh,
        out_type=jax.ShapeDtypeStruct((D, E), jnp.float32),
        scratch_types=[
            pltpu.MemorySpace.VMEM((16,), jnp.float32),
            pltpu.MemorySpace.VMEM((1, _NBUF), jnp.float32),
            pltpu.SemaphoreType.DMA,
        ],
    )
    def fill_kernel(w_hbm, o_hbm, wv, buf, sem):
        wid = lax.axis_index("s") * info.num_cores + lax.axis_index("c")
        for r in range(rows_per_w):
            row = rows_per_w * wid + r
            pltpu.sync_copy(w_hbm.at[pl.ds(row * 16, 16)], wv)
            splat = wv[...]

            @pl.loop(0, _NBUF, step=16)
            def _fill(i):
                buf[0, pl.ds(i, 16)] = splat

            @pl.loop(0, n_dma)
            def _fire(j):
                pltpu.make_async_copy(
                    buf,
                    o_hbm.at[pl.ds(row, 1), pl.ds(j * _NBUF, _NBUF)],
                    sem,
                ).start()

                @pl.when(j >= _WINDOW)
                def _():
                    pltpu.make_async_copy(
                        buf,
                        o_hbm.at[pl.ds(row, 1),
                                 pl.ds((j - _WINDOW) * _NBUF, _NBUF)],
                        sem,
                    ).wait()

            @pl.loop(max(n_dma - _WINDOW, 0), n_dma)
            def _drain(j):
                pltpu.make_async_copy(
                    buf,
                    o_hbm.at[pl.ds(row, 1), pl.ds(j * _NBUF, _NBUF)],
                    sem,
                ).wait()

    w_rep = jnp.repeat(W.reshape(D), 16)
    out_t = fill_kernel(w_rep)
    return out_t.T
